# baseline (speedup 1.0000x reference)
import functools

import jax
import jax.numpy as jnp
from jax import lax
from jax.experimental import pallas as pl
from jax.experimental.pallas import tpu as pltpu
from jax.experimental.pallas import tpu_sc as plsc

VOCAB = 1000
NE = 32
T = 8
BATCH = 4096
BF = BATCH * T

NC = 2
NS = 16
NW = NC * NS
BPW = BATCH // NW
L = 16


ROWS_PW = (T * NE) // NW


def _emb_body(tok_hbm, idx_hbm, xt_hbm, tok_v, idx_v, xtl_v):
    wid = lax.axis_index("s") * NC + lax.axis_index("c")
    t = (wid * ROWS_PW) // NE
    d0 = (wid * ROWS_PW) % NE
    pltpu.sync_copy(tok_hbm, tok_v)
    pltpu.sync_copy(idx_hbm.at[pl.ds(t * BATCH, BATCH)], idx_v)

    @plsc.parallel_loop(0, BATCH, step=L, unroll=8)
    def _(i):
        ti = idx_v[pl.ds(i, L)] + d0 * VOCAB
        for d in range(ROWS_PW):
            xtl_v[d, pl.ds(i, L)] = plsc.load_gather(tok_v, [ti + d * VOCAB])

    pltpu.sync_copy(xtl_v, xt_hbm.at[pl.ds(wid * ROWS_PW, ROWS_PW)])


@functools.cache
def _make_emb():
    return pl.kernel(
        _emb_body,
        out_type=jax.ShapeDtypeStruct((T * NE, BATCH), jnp.float32),
        mesh=plsc.VectorSubcoreMesh(
            core_axis_name="c", subcore_axis_name="s", num_cores=NC, num_subcores=NS
        ),
        scratch_types=[
            pltpu.VMEM((VOCAB * NE,), jnp.float32),
            pltpu.VMEM((BATCH,), jnp.int32),
            pltpu.VMEM((ROWS_PW, BATCH), jnp.float32),
        ],
        compiler_params=pltpu.CompilerParams(needs_layout_passes=False),
    )


BN = 2048


def _dot_bf16x2(wh, wl, x, dims):
    xh = x.astype(jnp.bfloat16)
    xl = (x - xh.astype(jnp.float32)).astype(jnp.bfloat16)
    kw = dict(precision=lax.Precision.DEFAULT, preferred_element_type=jnp.float32)
    return (
        lax.dot_general(wh, xh, dims, **kw)
        + lax.dot_general(wh, xl, dims, **kw)
        + lax.dot_general(wl, xh, dims, **kw)
    )


def _head_body(xt_ref, w_ref, pos_ref, b_ref, out_ref):
    w = w_ref[...]
    acc = lax.dot_general(
        w.astype(jnp.bfloat16),
        xt_ref[0].astype(jnp.bfloat16),
        (((1,), (0,)), ((), ())),
        preferred_element_type=jnp.float32,
    )
    pl_col = lax.dot_general(
        w,
        pos_ref[0],
        (((1,), (1,)), ((), ())),
        precision=lax.Precision.HIGHEST,
        preferred_element_type=jnp.float32,
    )
    out_ref[0] = acc + pl_col + b_ref[...]


BV = 200

_head = pl.pallas_call(
    _head_body,
    grid=(T, VOCAB // BV),
    in_specs=[
        pl.BlockSpec((1, NE, BATCH), lambda t, vb: (t, 0, 0)),
        pl.BlockSpec((BV, NE), lambda t, vb: (vb, 0)),
        pl.BlockSpec((1, 1, NE), lambda t, vb: (t, 0, 0)),
        pl.BlockSpec((BV, 1), lambda t, vb: (vb, 0)),
    ],
    out_specs=pl.BlockSpec((1, BV, BATCH), lambda t, vb: (t, vb, 0)),
    out_shape=jax.ShapeDtypeStruct((T, VOCAB, BATCH), jnp.float32),
)


def kernel(idx, tok_table, pos_table, W, b):
    idx_flat = idx.astype(jnp.int32).T.reshape(BF)
    tok_flat = tok_table.T.reshape(VOCAB * NE)
    xt = _make_emb()(tok_flat, idx_flat).reshape(T, NE, BATCH)
    ot = _head(xt, W, pos_table.reshape(T, 1, NE), b.reshape(VOCAB, 1))
    return jnp.transpose(ot, (2, 0, 1))


# speedup vs baseline: 1.1761x; 1.1761x over previous
import functools

import jax
import jax.numpy as jnp
from jax import lax
from jax.experimental import pallas as pl
from jax.experimental.pallas import tpu as pltpu
from jax.experimental.pallas import tpu_sc as plsc

VOCAB = 1000
NE = 32
T = 8
BATCH = 4096
BF = BATCH * T

NC = 2
NS = 16
NW = NC * NS
BPW = BATCH // NW
L = 16


ROWS_PW = (T * NE) // NW


def _emb_body(tok_hbm, idx_hbm, xt_hbm, tok_v, idx_v, xtl_v):
    wid = lax.axis_index("s") * NC + lax.axis_index("c")
    t = (wid * ROWS_PW) // NE
    d0 = (wid * ROWS_PW) % NE
    pltpu.sync_copy(tok_hbm, tok_v)
    pltpu.sync_copy(idx_hbm.at[pl.ds(t * BATCH, BATCH)], idx_v)

    @plsc.parallel_loop(0, BATCH, step=L, unroll=8)
    def _(i):
        ti = idx_v[pl.ds(i, L)] + d0 * VOCAB
        for d in range(ROWS_PW):
            xtl_v[d, pl.ds(i, L)] = plsc.load_gather(tok_v, [ti + d * VOCAB])

    pltpu.sync_copy(xtl_v, xt_hbm.at[pl.ds(wid * ROWS_PW, ROWS_PW)])


@functools.cache
def _make_emb():
    return pl.kernel(
        _emb_body,
        out_type=jax.ShapeDtypeStruct((T * NE, BATCH), jnp.float32),
        mesh=plsc.VectorSubcoreMesh(
            core_axis_name="c", subcore_axis_name="s", num_cores=NC, num_subcores=NS
        ),
        scratch_types=[
            pltpu.VMEM((VOCAB * NE,), jnp.float32),
            pltpu.VMEM((BATCH,), jnp.int32),
            pltpu.VMEM((ROWS_PW, BATCH), jnp.float32),
        ],
        compiler_params=pltpu.CompilerParams(needs_layout_passes=False),
    )


BN = 2048


def _dot_bf16x2(wh, wl, x, dims):
    xh = x.astype(jnp.bfloat16)
    xl = (x - xh.astype(jnp.float32)).astype(jnp.bfloat16)
    kw = dict(precision=lax.Precision.DEFAULT, preferred_element_type=jnp.float32)
    return (
        lax.dot_general(wh, xh, dims, **kw)
        + lax.dot_general(wh, xl, dims, **kw)
        + lax.dot_general(wl, xh, dims, **kw)
    )


def _head_body(xt_ref, w_ref, pos_ref, b_ref, out_ref):
    w = w_ref[...]
    acc = lax.dot_general(
        w.astype(jnp.bfloat16),
        xt_ref[0].astype(jnp.bfloat16),
        (((1,), (0,)), ((), ())),
        preferred_element_type=jnp.float32,
    )
    pl_col = lax.dot_general(
        w,
        pos_ref[0],
        (((1,), (1,)), ((), ())),
        precision=lax.Precision.HIGHEST,
        preferred_element_type=jnp.float32,
    )
    out_ref[0] = acc + pl_col + b_ref[...]


_head = pl.pallas_call(
    _head_body,
    grid=(T, BATCH // BN),
    in_specs=[
        pl.BlockSpec((1, NE, BN), lambda t, nb: (t, 0, nb)),
        pl.BlockSpec((VOCAB, NE), lambda t, nb: (0, 0)),
        pl.BlockSpec((1, 1, NE), lambda t, nb: (t, 0, 0)),
        pl.BlockSpec((VOCAB, 1), lambda t, nb: (0, 0)),
    ],
    out_specs=pl.BlockSpec((1, VOCAB, BN), lambda t, nb: (t, 0, nb)),
    out_shape=jax.ShapeDtypeStruct((T, VOCAB, BATCH), jnp.float32),
)


def kernel(idx, tok_table, pos_table, W, b):
    idx_flat = idx.astype(jnp.int32).T.reshape(BF)
    tok_flat = tok_table.T.reshape(VOCAB * NE)
    xt = _make_emb()(tok_flat, idx_flat).reshape(T, NE, BATCH)
    ot = _head(xt, W, pos_table.reshape(T, 1, NE), b.reshape(VOCAB, 1))
    return jnp.transpose(ot, (2, 0, 1))


# speedup vs baseline: 1.1807x; 1.0039x over previous
import functools

import jax
import jax.numpy as jnp
from jax import lax
from jax.experimental import pallas as pl
from jax.experimental.pallas import tpu as pltpu
from jax.experimental.pallas import tpu_sc as plsc

VOCAB = 1000
NE = 32
T = 8
BATCH = 4096
BF = BATCH * T

NC = 2
NS = 16
NW = NC * NS
BPW = BATCH // NW
L = 16


ROWS_PW = (T * NE) // NW


def _emb_body(tok_hbm, idx_hbm, xt_hbm, tok_v, idx_v, xtl_v):
    wid = lax.axis_index("s") * NC + lax.axis_index("c")
    t = (wid * ROWS_PW) // NE
    d0 = (wid * ROWS_PW) % NE
    pltpu.sync_copy(tok_hbm, tok_v)
    pltpu.sync_copy(idx_hbm.at[pl.ds(t * BATCH, BATCH)], idx_v)

    @plsc.parallel_loop(0, BATCH, step=L, unroll=8)
    def _(i):
        ti = idx_v[pl.ds(i, L)] + d0 * VOCAB
        for d in range(ROWS_PW):
            xtl_v[d, pl.ds(i, L)] = plsc.load_gather(tok_v, [ti + d * VOCAB])

    pltpu.sync_copy(xtl_v, xt_hbm.at[pl.ds(wid * ROWS_PW, ROWS_PW)])


@functools.cache
def _make_emb():
    return pl.kernel(
        _emb_body,
        out_type=jax.ShapeDtypeStruct((T * NE, BATCH), jnp.float32),
        mesh=plsc.VectorSubcoreMesh(
            core_axis_name="c", subcore_axis_name="s", num_cores=NC, num_subcores=NS
        ),
        scratch_types=[
            pltpu.VMEM((VOCAB * NE,), jnp.float32),
            pltpu.VMEM((BATCH,), jnp.int32),
            pltpu.VMEM((ROWS_PW, BATCH), jnp.float32),
        ],
        compiler_params=pltpu.CompilerParams(needs_layout_passes=False),
    )


BN = 2048


def _head_body(xt_ref, w_ref, pos_ref, b_ref, out_ref):
    w = w_ref[...]
    acc = lax.dot_general(
        w.astype(jnp.bfloat16),
        xt_ref[0].astype(jnp.bfloat16),
        (((1,), (0,)), ((), ())),
        preferred_element_type=jnp.float32,
    )
    pl_col = lax.dot_general(
        w,
        pos_ref[0],
        (((1,), (1,)), ((), ())),
        precision=lax.Precision.HIGHEST,
        preferred_element_type=jnp.float32,
    )
    out_ref[0] = acc + pl_col + b_ref[...]


_head = pl.pallas_call(
    _head_body,
    grid=(T, BATCH // BN),
    in_specs=[
        pl.BlockSpec((1, NE, BN), lambda t, nb: (t, 0, nb)),
        pl.BlockSpec((VOCAB, NE), lambda t, nb: (0, 0)),
        pl.BlockSpec((1, 1, NE), lambda t, nb: (t, 0, 0)),
        pl.BlockSpec((VOCAB, 1), lambda t, nb: (0, 0)),
    ],
    out_specs=pl.BlockSpec((1, VOCAB, BN), lambda t, nb: (t, 0, nb)),
    out_shape=jax.ShapeDtypeStruct((T, VOCAB, BATCH), jnp.float32),
)


def kernel(idx, tok_table, pos_table, W, b):
    idx_flat = idx.astype(jnp.int32).T.reshape(BF)
    tok_flat = tok_table.T.reshape(VOCAB * NE)
    xt = _make_emb()(tok_flat, idx_flat).reshape(T, NE, BATCH)
    ot = _head(xt, W, pos_table.reshape(T, 1, NE), b.reshape(VOCAB, 1))
    return jnp.transpose(ot, (2, 0, 1))
